# baseline (speedup 1.0000x reference)
import functools

import jax
import jax.numpy as jnp
from jax import lax
from jax.experimental import pallas as pl
from jax.experimental.pallas import tpu as pltpu
from jax.experimental.pallas import tpu_sc as plsc

N = 10000
E = 320000
EF = E + N
D = 128
NP = 10240
TRASH = N
NTILES = 32
CH_G = 128
NCHUNK = 81
EP = NTILES * NCHUNK * CH_G
PER_TILE = NCHUNK * CH_G
SLICE = NP // 16

_mesh = plsc.VectorSubcoreMesh(core_axis_name="c", subcore_axis_name="s")
_sc_params = pltpu.CompilerParams(needs_layout_passes=False)


def _iota16():
    return lax.iota(jnp.int32, 16)


def _splat(v, dtype=jnp.int32):
    return jnp.full((16,), v, dtype=dtype)


_GDN = lax.GatherDimensionNumbers(offset_dims=(), collapsed_slice_dims=(0,),
                                  start_index_map=(0,))


def _perm(v, idx):
    return lax.gather(v, idx[:, None], _GDN, (1,),
                      mode=lax.GatherScatterMode.PROMISE_IN_BOUNDS)


def _lane_sum(v, iota):
    for sh in (8, 4, 2, 1):
        v = v + _perm(v, iota ^ sh)
    return v


def _dst_eff(src16, dst16, e16):
    valid = jnp.where(e16 < E, src16 != dst16, e16 < EF)
    return jnp.where(valid, dst16, _splat(TRASH))


def _pass_a_body(h_hbm, sd_hbm, al_hbm, ar_hbm,
                 alpha_hbm, m_hbm,
                 al_v, ar_v, m_v, idx_v, srows_v, drows_v,
                 alpha_v, red_v, mslice_v, m_sh, ss0, ss1, sd0, sd1, sa0, sa1):
    cid = lax.axis_index("c")
    sid = lax.axis_index("s")
    wid = sid * 2 + cid
    iota = _iota16()
    sems = ((ss0, sd0), (ss1, sd1))
    a_sems = (sa0, sa1)
    n0 = 67
    n1 = 2 * NCHUNK - n0
    ncha = jnp.where(cid == 0, n0, n1)
    start = jnp.where(cid == 0, sid * n0, 16 * n0 + sid * n1)

    pltpu.sync_copy(al_hbm, al_v)
    pltpu.sync_copy(ar_hbm, ar_v)

    def init_body(w, _):
        m_v[pl.ds(w * 16, 16)] = _splat(-3e38, jnp.float32)
        return 0
    lax.fori_loop(0, NP // 16, init_body, 0)

    def issue(c, b):
        pltpu.sync_copy(sd_hbm.at[start + c], idx_v.at[b])
        pltpu.make_async_copy(h_hbm.at[idx_v.at[b, 0]], srows_v.at[b],
                              sems[b][0]).start()
        pltpu.make_async_copy(h_hbm.at[idx_v.at[b, 1]], drows_v.at[b],
                              sems[b][1]).start()

    def wait(b):
        pltpu.make_async_copy(h_hbm.at[idx_v.at[b, 0]], srows_v.at[b],
                              sems[b][0]).wait()
        pltpu.make_async_copy(h_hbm.at[idx_v.at[b, 1]], drows_v.at[b],
                              sems[b][1]).wait()

    def compute(c, b):
        base = (start + c) * CH_G

        def dot_body(l, dots):
            out = []
            for g in range(CH_G // 16):
                r = l + g * 16
                acc = srows_v[b, r, pl.ds(0, 16)] * drows_v[b, r, pl.ds(0, 16)]
                for j in range(1, D // 16):
                    sl = pl.ds(j * 16, 16)
                    acc = acc + srows_v[b, r, sl] * drows_v[b, r, sl]
                out.append(jnp.where(iota == l, _lane_sum(acc, iota),
                                     dots[g]))
            return tuple(out)
        accs = lax.fori_loop(0, 16, dot_body,
                             tuple(jnp.zeros((16,), jnp.float32)
                                   for _ in range(CH_G // 16)))

        for g in range(CH_G // 16):
            src16 = idx_v[b, 0, pl.ds(g * 16, 16)]
            dst16 = idx_v[b, 1, pl.ds(g * 16, 16)]
            e16 = iota + (base + g * 16)
            deff = _dst_eff(src16, dst16, e16)
            logits = accs[g]
            sig = 1.0 / (1.0 + jnp.exp(-logits))
            raw = (plsc.load_gather(al_v, [src16])
                   + plsc.load_gather(ar_v, [dst16]))
            a = raw * sig
            a = jnp.where(a >= 0.0, a, 0.2 * a)
            alpha_v[b, pl.ds(g * 16, 16)] = a

            def mx_cond(pending):
                return jnp.any(pending)

            def mx_body(pending):
                cur = plsc.load_gather(m_v, [deff])
                nxt = pending & (a > cur)
                plsc.store_scatter(m_v, [deff], a, mask=nxt)
                return nxt
            lax.while_loop(mx_cond, mx_body, _splat(True, jnp.bool_))

        pltpu.make_async_copy(alpha_v.at[b], alpha_hbm.at[pl.ds(base, CH_G)],
                              a_sems[b]).start()

    def alpha_wait(c, b):
        base = (start + c) * CH_G
        pltpu.make_async_copy(alpha_v.at[b], alpha_hbm.at[pl.ds(base, CH_G)],
                              a_sems[b]).wait()

    issue(0, 0)

    def piped(i, _):
        c0 = i * 2
        issue(c0 + 1, 1)
        wait(0)
        lax.cond(i > 0, lambda: alpha_wait(c0 - 2, 0), lambda: None)
        compute(c0, 0)
        issue(c0 + 2, 0)
        wait(1)
        lax.cond(i > 0, lambda: alpha_wait(c0 - 1, 1), lambda: None)
        compute(c0 + 1, 1)
        return 0
    lax.fori_loop(0, (ncha - 1) // 2, piped, 0)
    wait(0)
    alpha_wait(ncha - 3, 0)
    compute(ncha - 1, 0)
    alpha_wait(ncha - 2, 1)
    alpha_wait(ncha - 1, 0)

    pltpu.sync_copy(m_v, m_sh.at[sid])
    plsc.subcore_barrier()
    pltpu.sync_copy(m_sh.at[:, pl.ds(sid * SLICE, SLICE)], red_v)
    for w in range(SLICE // 16):
        acc = red_v[0, pl.ds(w * 16, 16)]
        for t in range(1, 16):
            acc = jnp.maximum(acc, red_v[t, pl.ds(w * 16, 16)])
        mslice_v[pl.ds(w * 16, 16)] = acc
    pltpu.sync_copy(mslice_v, m_hbm.at[cid, pl.ds(sid * SLICE, SLICE)])


@functools.partial(
    pl.kernel,
    out_type=(jax.ShapeDtypeStruct((EP,), jnp.float32),
              jax.ShapeDtypeStruct((2, NP), jnp.float32)),
    mesh=_mesh,
    compiler_params=_sc_params,
    scratch_types=[
        pltpu.VMEM((N,), jnp.float32),
        pltpu.VMEM((N,), jnp.float32),
        pltpu.VMEM((NP,), jnp.float32),
        pltpu.VMEM((2, 2, CH_G), jnp.int32),
        pltpu.VMEM((2, CH_G, D), jnp.float32),
        pltpu.VMEM((2, CH_G, D), jnp.float32),
        pltpu.VMEM((2, CH_G), jnp.float32),
        pltpu.VMEM((16, SLICE), jnp.float32),
        pltpu.VMEM((SLICE,), jnp.float32),
        pltpu.VMEM_SHARED((16, NP), jnp.float32),
        pltpu.SemaphoreType.DMA,
        pltpu.SemaphoreType.DMA,
        pltpu.SemaphoreType.DMA,
        pltpu.SemaphoreType.DMA,
        pltpu.SemaphoreType.DMA,
        pltpu.SemaphoreType.DMA,
    ],
)
def _pass_a(h_hbm, sd_hbm, al_hbm, ar_hbm, alpha_hbm, m_hbm,
            al_v, ar_v, m_v, idx_v, srows_v, drows_v, alpha_v,
            red_v, mslice_v, m_sh, ss0, ss1, sd0, sd1, sa0, sa1):
    _pass_a_body(h_hbm, sd_hbm, al_hbm, ar_hbm, alpha_hbm, m_hbm,
                 al_v, ar_v, m_v, idx_v, srows_v, drows_v, alpha_v,
                 red_v, mslice_v, m_sh, ss0, ss1, sd0, sd1, sa0, sa1)


def _pass_b_body(h_hbm, sd_hbm, alpha_hbm, m_hbm,
                 den_hbm, num_hbm,
                 m_v, idx_v, deff_v, srows_v,
                 alpha_v, exc_v, zs_v, den_sh, num_sh, ss0, ss1, sc0, sc1):
    cid = lax.axis_index("c")
    sid = lax.axis_index("s")
    wid = sid * 2 + cid
    iota = _iota16()
    sems = (ss0, ss1)
    sc_sems = (sc0, sc1)

    pltpu.sync_copy(m_hbm, m_v)

    def zrow_body(r, _):
        for j in range(D // 16):
            srows_v[0, r, pl.ds(j * 16, 16)] = jnp.zeros((16,), jnp.float32)
        return 0
    lax.fori_loop(0, CH_G, zrow_body, 0)
    for bb in range(SLICE // CH_G):
        pltpu.sync_copy(srows_v.at[0],
                        num_sh.at[pl.ds(sid * SLICE + bb * CH_G, CH_G)])

    def zden_body(w, _):
        zs_v[pl.ds(w * 16, 16)] = jnp.zeros((16,), jnp.float32)
        return 0
    lax.fori_loop(0, SLICE // 16, zden_body, 0)
    pltpu.sync_copy(zs_v, den_sh.at[pl.ds(sid * SLICE, SLICE)])
    plsc.subcore_barrier()

    def issue(c, b):
        base = wid * PER_TILE + c * CH_G
        pltpu.sync_copy(sd_hbm.at[wid * NCHUNK + c], idx_v.at[b])
        pltpu.sync_copy(alpha_hbm.at[pl.ds(base, CH_G)], alpha_v.at[b])
        pltpu.make_async_copy(h_hbm.at[idx_v.at[b, 0]], srows_v.at[b],
                              sems[b]).start()

    def wait(b):
        pltpu.make_async_copy(h_hbm.at[idx_v.at[b, 0]], srows_v.at[b],
                              sems[b]).wait()

    def compute(c, b):
        base = wid * PER_TILE + c * CH_G
        for g in range(CH_G // 16):
            src16 = idx_v[b, 0, pl.ds(g * 16, 16)]
            dst16 = idx_v[b, 1, pl.ds(g * 16, 16)]
            e16 = iota + (base + g * 16)
            deff = _dst_eff(src16, dst16, e16)
            deff_v[b, pl.ds(g * 16, 16)] = deff
            a16 = alpha_v[b, pl.ds(g * 16, 16)]
            ex = jnp.exp(a16 - plsc.load_gather(m_v, [deff]))
            exc_v[pl.ds(g * 16, 16)] = ex
            for l in range(16):
                exb = _perm(ex, _splat(l))
                r = g * 16 + l
                for j in range(D // 16):
                    sl = pl.ds(j * 16, 16)
                    srows_v[b, r, sl] = srows_v[b, r, sl] * exb

        pltpu.make_async_copy(srows_v.at[b], num_sh.at[deff_v.at[b]],
                              sc_sems[b]).start(add=True)
        pltpu.sync_copy(exc_v, den_sh.at[deff_v.at[b]], add=True)

    def scat_wait(b):
        pltpu.make_async_copy(srows_v.at[b], num_sh.at[deff_v.at[b]],
                              sc_sems[b]).wait()

    issue(0, 0)

    def piped(i, _):
        c0 = i * 2
        issue(c0 + 1, 1)
        wait(0)
        lax.cond(i > 0, lambda: scat_wait(0), lambda: None)
        compute(c0, 0)
        issue(c0 + 2, 0)
        wait(1)
        lax.cond(i > 0, lambda: scat_wait(1), lambda: None)
        compute(c0 + 1, 1)
        return 0
    lax.fori_loop(0, (NCHUNK - 1) // 2, piped, 0)
    wait(0)
    scat_wait(0)
    compute(NCHUNK - 1, 0)
    scat_wait(1)
    scat_wait(0)
    plsc.subcore_barrier()

    pltpu.sync_copy(den_sh.at[pl.ds(sid * SLICE, SLICE)],
                    den_hbm.at[cid, pl.ds(sid * SLICE, SLICE)])
    pltpu.sync_copy(num_sh.at[pl.ds(sid * SLICE, SLICE)],
                    num_hbm.at[cid, pl.ds(sid * SLICE, SLICE)])


@functools.partial(
    pl.kernel,
    out_type=(jax.ShapeDtypeStruct((2, NP), jnp.float32),
              jax.ShapeDtypeStruct((2, NP, D), jnp.float32)),
    mesh=_mesh,
    compiler_params=_sc_params,
    scratch_types=[
        pltpu.VMEM((NP,), jnp.float32),
        pltpu.VMEM((2, 2, CH_G), jnp.int32),
        pltpu.VMEM((2, CH_G), jnp.int32),
        pltpu.VMEM((2, CH_G, D), jnp.float32),
        pltpu.VMEM((2, CH_G), jnp.float32),
        pltpu.VMEM((CH_G,), jnp.float32),
        pltpu.VMEM((SLICE,), jnp.float32),
        pltpu.VMEM_SHARED((NP,), jnp.float32),
        pltpu.VMEM_SHARED((NP, D), jnp.float32),
        pltpu.SemaphoreType.DMA,
        pltpu.SemaphoreType.DMA,
        pltpu.SemaphoreType.DMA,
        pltpu.SemaphoreType.DMA,
    ],
)
def _pass_b(h_hbm, sd_hbm, alpha_hbm, m_hbm, den_hbm, num_hbm,
            m_v, idx_v, deff_v, srows_v, alpha_v,
            exc_v, zs_v, den_sh, num_sh, ss0, ss1, sc0, sc1):
    _pass_b_body(h_hbm, sd_hbm, alpha_hbm, m_hbm, den_hbm, num_hbm,
                 m_v, idx_v, deff_v, srows_v, alpha_v,
                 exc_v, zs_v, den_sh, num_sh, ss0, ss1, sc0, sc1)


def _tcmax_body(m_ref, o_ref):
    o_ref[...] = jnp.maximum(m_ref[0], m_ref[1])


def _tc_max(m):
    return pl.pallas_call(
        _tcmax_body,
        out_shape=jax.ShapeDtypeStruct((NP,), jnp.float32),
    )(m)


_RB = 1000


def _mmT(a, w):
    return lax.dot_general(a, w, (((1,), (1,)), ((), ())),
                           preferred_element_type=jnp.float32)


def _tc1_body(x_ref, w_ref, attl_ref, attr_ref, h_ref, al_ref, ar_ref):
    h = _mmT(x_ref[...], w_ref[...])
    h_ref[...] = h
    al_ref[...] = _mmT(h, attl_ref[...])
    ar_ref[...] = _mmT(h, attr_ref[...])


def _tc_dense1(x, W, att_l, att_r):
    return pl.pallas_call(
        _tc1_body,
        grid=(N // _RB,),
        in_specs=[
            pl.BlockSpec((_RB, D), lambda i: (i, 0)),
            pl.BlockSpec((D, D), lambda i: (0, 0)),
            pl.BlockSpec((1, D), lambda i: (0, 0)),
            pl.BlockSpec((1, D), lambda i: (0, 0)),
        ],
        out_specs=[
            pl.BlockSpec((_RB, D), lambda i: (i, 0)),
            pl.BlockSpec((_RB, 1), lambda i: (i, 0)),
            pl.BlockSpec((_RB, 1), lambda i: (i, 0)),
        ],
        out_shape=[
            jax.ShapeDtypeStruct((N, D), jnp.float32),
            jax.ShapeDtypeStruct((N, 1), jnp.float32),
            jax.ShapeDtypeStruct((N, 1), jnp.float32),
        ],
    )(x, W, att_l.reshape(1, D), att_r.reshape(1, D))


def _bn(x, g, b, m, v, eps=1e-5):
    return (x - m) / jnp.sqrt(v + eps) * g + b


def _tc2_body(num_ref, den_ref, b_ref, g_ref, bb_ref, m_ref, v_ref,
              w_ref, attl_ref, attr_ref, h_ref, al_ref, ar_ref):
    num = num_ref[0] + num_ref[1]
    den = den_ref[0, :, 0] + den_ref[1, :, 0]
    out = num / (den[:, None] + 1e-16) + b_ref[...]
    out = _bn(out, g_ref[...], bb_ref[...], m_ref[...], v_ref[...])
    hin = jnp.maximum(out, 0.0)
    h = _mmT(hin, w_ref[...])
    h_ref[...] = h
    al_ref[...] = _mmT(h, attl_ref[...])
    ar_ref[...] = _mmT(h, attr_ref[...])


def _tc_dense2(num, den, bias, bn_g, bn_b, bn_m, bn_v, W, att_l, att_r):
    vec = lambda a: a.reshape(1, D)
    return pl.pallas_call(
        _tc2_body,
        grid=(N // _RB,),
        in_specs=[
            pl.BlockSpec((2, _RB, D), lambda i: (0, i, 0)),
            pl.BlockSpec((2, _RB, 1), lambda i: (0, i, 0)),
        ] + [pl.BlockSpec((1, D), lambda i: (0, 0))] * 5 + [
            pl.BlockSpec((D, D), lambda i: (0, 0)),
            pl.BlockSpec((1, D), lambda i: (0, 0)),
            pl.BlockSpec((1, D), lambda i: (0, 0)),
        ],
        out_specs=[
            pl.BlockSpec((_RB, D), lambda i: (i, 0)),
            pl.BlockSpec((_RB, 1), lambda i: (i, 0)),
            pl.BlockSpec((_RB, 1), lambda i: (i, 0)),
        ],
        out_shape=[
            jax.ShapeDtypeStruct((N, D), jnp.float32),
            jax.ShapeDtypeStruct((N, 1), jnp.float32),
            jax.ShapeDtypeStruct((N, 1), jnp.float32),
        ],
    )(num, den.reshape(2, NP, 1), vec(bias), vec(bn_g), vec(bn_b),
      vec(bn_m), vec(bn_v), W, att_l.reshape(1, D), att_r.reshape(1, D))


def _tc3_body(num_ref, den_ref, b_ref, g2_ref, b2_ref, m2_ref, v2_ref,
              f1w_ref, f1b_ref, g3_ref, b3_ref, m3_ref, v3_ref,
              f2w_ref, f2b_ref, g4_ref, b4_ref, m4_ref, v4_ref,
              f3w_ref, f3b_ref, o_ref):
    num = num_ref[0] + num_ref[1]
    den = den_ref[0, :, 0] + den_ref[1, :, 0]
    out = num / (den[:, None] + 1e-16) + b_ref[...]
    h = jnp.maximum(_bn(out, g2_ref[...], b2_ref[...], m2_ref[...],
                        v2_ref[...]), 0.0)
    h = jnp.maximum(_bn(_mmT(h, f1w_ref[...]) + f1b_ref[...],
                        g3_ref[...], b3_ref[...], m3_ref[...], v3_ref[...]),
                    0.0)
    h = jnp.maximum(_bn(_mmT(h, f2w_ref[...]) + f2b_ref[...],
                        g4_ref[...], b4_ref[...], m4_ref[...], v4_ref[...]),
                    0.0)
    o_ref[...] = _mmT(h, f3w_ref[...]) + f3b_ref[...]


def _tc_dense3(num, den, b2, bn2_g, bn2_b, bn2_m, bn2_v,
               fc1_w, fc1_b, bn3_g, bn3_b, bn3_m, bn3_v,
               fc2_w, fc2_b, bn4_g, bn4_b, bn4_m, bn4_v, fc3_w, fc3_b):
    vec = lambda a: a.reshape(1, -1)
    return pl.pallas_call(
        _tc3_body,
        grid=(N // _RB,),
        in_specs=[
            pl.BlockSpec((2, _RB, D), lambda i: (0, i, 0)),
            pl.BlockSpec((2, _RB, 1), lambda i: (0, i, 0)),
        ] + [pl.BlockSpec((1, D), lambda i: (0, 0))] * 5 + [
            pl.BlockSpec((D, D), lambda i: (0, 0)),
        ] + [pl.BlockSpec((1, D), lambda i: (0, 0))] * 5 + [
            pl.BlockSpec((D, D), lambda i: (0, 0)),
        ] + [pl.BlockSpec((1, D), lambda i: (0, 0))] * 5 + [
            pl.BlockSpec((2, D), lambda i: (0, 0)),
            pl.BlockSpec((1, 2), lambda i: (0, 0)),
        ],
        out_specs=[pl.BlockSpec((_RB, 2), lambda i: (i, 0))],
        out_shape=[jax.ShapeDtypeStruct((N, 2), jnp.float32)],
    )(num, den.reshape(2, NP, 1), vec(b2), vec(bn2_g), vec(bn2_b),
      vec(bn2_m), vec(bn2_v), fc1_w, vec(fc1_b), vec(bn3_g), vec(bn3_b),
      vec(bn3_m), vec(bn3_v), fc2_w, vec(fc2_b), vec(bn4_g), vec(bn4_b),
      vec(bn4_m), vec(bn4_v), fc3_w, vec(fc3_b))[0]


def kernel(x, edge_index, W1, att_l1, att_r1, b1, bn1_g, bn1_b, bn1_m, bn1_v,
           W2, att_l2, att_r2, b2, bn2_g, bn2_b, bn2_m, bn2_v,
           fc1_w, fc1_b, bn3_g, bn3_b, bn3_m, bn3_v,
           fc2_w, fc2_b, bn4_g, bn4_b, bn4_m, bn4_v, fc3_w, fc3_b):
    loops = jnp.arange(N, dtype=jnp.int32)
    pad = jnp.zeros((EP - EF,), jnp.int32)
    src = jnp.concatenate([edge_index[0], loops, pad])
    dst = jnp.concatenate([edge_index[1], loops, pad])
    sd = jnp.stack([src.reshape(-1, CH_G), dst.reshape(-1, CH_G)], axis=1)

    h1, al1, ar1 = _tc_dense1(x, W1, att_l1, att_r1)
    alpha1, m1 = _pass_a(h1, sd, al1.reshape(N), ar1.reshape(N))
    den1, num1 = _pass_b(h1, sd, alpha1, _tc_max(m1))
    h2, al2, ar2 = _tc_dense2(num1, den1, b1, bn1_g, bn1_b, bn1_m, bn1_v,
                              W2, att_l2, att_r2)
    alpha2, m2 = _pass_a(h2, sd, al2.reshape(N), ar2.reshape(N))
    den2, num2 = _pass_b(h2, sd, alpha2, _tc_max(m2))
    return _tc_dense3(num2, den2, b2, bn2_g, bn2_b, bn2_m, bn2_v,
                      fc1_w, fc1_b, bn3_g, bn3_b, bn3_m, bn3_v,
                      fc2_w, fc2_b, bn4_g, bn4_b, bn4_m, bn4_v,
                      fc3_w, fc3_b)


# speedup vs baseline: 1.0823x; 1.0823x over previous
import functools

import jax
import jax.numpy as jnp
from jax import lax
from jax.experimental import pallas as pl
from jax.experimental.pallas import tpu as pltpu
from jax.experimental.pallas import tpu_sc as plsc

N = 10000
E = 320000
EF = E + N
D = 128
NP = 10240
TRASH = N
NTILES = 32
CH_G = 128
NCHUNK = 81
EP = NTILES * NCHUNK * CH_G
PER_TILE = NCHUNK * CH_G
SLICE = NP // 16

_mesh = plsc.VectorSubcoreMesh(core_axis_name="c", subcore_axis_name="s")
_sc_params = pltpu.CompilerParams(needs_layout_passes=False)


def _iota16():
    return lax.iota(jnp.int32, 16)


def _splat(v, dtype=jnp.int32):
    return jnp.full((16,), v, dtype=dtype)


_GDN = lax.GatherDimensionNumbers(offset_dims=(), collapsed_slice_dims=(0,),
                                  start_index_map=(0,))


def _perm(v, idx):
    return lax.gather(v, idx[:, None], _GDN, (1,),
                      mode=lax.GatherScatterMode.PROMISE_IN_BOUNDS)


def _lane_sum(v, iota):
    for sh in (8, 4, 2, 1):
        v = v + _perm(v, iota ^ sh)
    return v


def _dst_eff(src16, dst16, e16):
    valid = jnp.where(e16 < E, src16 != dst16, e16 < EF)
    return jnp.where(valid, dst16, _splat(TRASH))


def _pass_a_body(h_hbm, sd_hbm, al_hbm, ar_hbm,
                 alpha_hbm, m_hbm,
                 al_v, ar_v, m_v, idx_v, srows_v, drows_v,
                 alpha_v, red_v, mslice_v, m_sh, ss0, ss1, sd0, sd1, sa0, sa1):
    cid = lax.axis_index("c")
    sid = lax.axis_index("s")
    wid = sid * 2 + cid
    iota = _iota16()
    sems = ((ss0, sd0), (ss1, sd1))
    a_sems = (sa0, sa1)
    n0 = 95
    n1 = 2 * NCHUNK - n0
    ncha = jnp.where(cid == 0, n0, n1)
    start = jnp.where(cid == 0, sid * n0, 16 * n0 + sid * n1)

    pltpu.sync_copy(al_hbm, al_v)
    pltpu.sync_copy(ar_hbm, ar_v)

    def init_body(w, _):
        m_v[pl.ds(w * 16, 16)] = _splat(-3e38, jnp.float32)
        return 0
    lax.fori_loop(0, NP // 16, init_body, 0)

    def issue(c, b):
        pltpu.sync_copy(sd_hbm.at[start + c], idx_v.at[b])
        pltpu.make_async_copy(h_hbm.at[idx_v.at[b, 0]], srows_v.at[b],
                              sems[b][0]).start()
        pltpu.make_async_copy(h_hbm.at[idx_v.at[b, 1]], drows_v.at[b],
                              sems[b][1]).start()

    def wait(b):
        pltpu.make_async_copy(h_hbm.at[idx_v.at[b, 0]], srows_v.at[b],
                              sems[b][0]).wait()
        pltpu.make_async_copy(h_hbm.at[idx_v.at[b, 1]], drows_v.at[b],
                              sems[b][1]).wait()

    def compute(c, b):
        base = (start + c) * CH_G

        def dot_body(l, dots):
            out = []
            for g in range(CH_G // 16):
                r = l + g * 16
                acc = srows_v[b, r, pl.ds(0, 16)] * drows_v[b, r, pl.ds(0, 16)]
                for j in range(1, D // 16):
                    sl = pl.ds(j * 16, 16)
                    acc = acc + srows_v[b, r, sl] * drows_v[b, r, sl]
                out.append(jnp.where(iota == l, _lane_sum(acc, iota),
                                     dots[g]))
            return tuple(out)
        accs = lax.fori_loop(0, 16, dot_body,
                             tuple(jnp.zeros((16,), jnp.float32)
                                   for _ in range(CH_G // 16)))

        for g in range(CH_G // 16):
            src16 = idx_v[b, 0, pl.ds(g * 16, 16)]
            dst16 = idx_v[b, 1, pl.ds(g * 16, 16)]
            e16 = iota + (base + g * 16)
            deff = _dst_eff(src16, dst16, e16)
            logits = accs[g]
            sig = 1.0 / (1.0 + jnp.exp(-logits))
            raw = (plsc.load_gather(al_v, [src16])
                   + plsc.load_gather(ar_v, [dst16]))
            a = raw * sig
            a = jnp.where(a >= 0.0, a, 0.2 * a)
            alpha_v[b, pl.ds(g * 16, 16)] = a

            def mx_cond(pending):
                return jnp.any(pending)

            def mx_body(pending):
                cur = plsc.load_gather(m_v, [deff])
                nxt = pending & (a > cur)
                plsc.store_scatter(m_v, [deff], a, mask=nxt)
                return nxt
            lax.while_loop(mx_cond, mx_body, _splat(True, jnp.bool_))

        pltpu.make_async_copy(alpha_v.at[b], alpha_hbm.at[pl.ds(base, CH_G)],
                              a_sems[b]).start()

    def alpha_wait(c, b):
        base = (start + c) * CH_G
        pltpu.make_async_copy(alpha_v.at[b], alpha_hbm.at[pl.ds(base, CH_G)],
                              a_sems[b]).wait()

    issue(0, 0)

    def piped(i, _):
        c0 = i * 2
        issue(c0 + 1, 1)
        wait(0)
        lax.cond(i > 0, lambda: alpha_wait(c0 - 2, 0), lambda: None)
        compute(c0, 0)
        issue(c0 + 2, 0)
        wait(1)
        lax.cond(i > 0, lambda: alpha_wait(c0 - 1, 1), lambda: None)
        compute(c0 + 1, 1)
        return 0
    lax.fori_loop(0, (ncha - 1) // 2, piped, 0)
    wait(0)
    alpha_wait(ncha - 3, 0)
    compute(ncha - 1, 0)
    alpha_wait(ncha - 2, 1)
    alpha_wait(ncha - 1, 0)

    pltpu.sync_copy(m_v, m_sh.at[sid])
    plsc.subcore_barrier()
    pltpu.sync_copy(m_sh.at[:, pl.ds(sid * SLICE, SLICE)], red_v)
    for w in range(SLICE // 16):
        acc = red_v[0, pl.ds(w * 16, 16)]
        for t in range(1, 16):
            acc = jnp.maximum(acc, red_v[t, pl.ds(w * 16, 16)])
        mslice_v[pl.ds(w * 16, 16)] = acc
    pltpu.sync_copy(mslice_v, m_hbm.at[cid, pl.ds(sid * SLICE, SLICE)])


@functools.partial(
    pl.kernel,
    out_type=(jax.ShapeDtypeStruct((EP,), jnp.float32),
              jax.ShapeDtypeStruct((2, NP), jnp.float32)),
    mesh=_mesh,
    compiler_params=_sc_params,
    scratch_types=[
        pltpu.VMEM((N,), jnp.float32),
        pltpu.VMEM((N,), jnp.float32),
        pltpu.VMEM((NP,), jnp.float32),
        pltpu.VMEM((2, 2, CH_G), jnp.int32),
        pltpu.VMEM((2, CH_G, D), jnp.float32),
        pltpu.VMEM((2, CH_G, D), jnp.float32),
        pltpu.VMEM((2, CH_G), jnp.float32),
        pltpu.VMEM((16, SLICE), jnp.float32),
        pltpu.VMEM((SLICE,), jnp.float32),
        pltpu.VMEM_SHARED((16, NP), jnp.float32),
        pltpu.SemaphoreType.DMA,
        pltpu.SemaphoreType.DMA,
        pltpu.SemaphoreType.DMA,
        pltpu.SemaphoreType.DMA,
        pltpu.SemaphoreType.DMA,
        pltpu.SemaphoreType.DMA,
    ],
)
def _pass_a(h_hbm, sd_hbm, al_hbm, ar_hbm, alpha_hbm, m_hbm,
            al_v, ar_v, m_v, idx_v, srows_v, drows_v, alpha_v,
            red_v, mslice_v, m_sh, ss0, ss1, sd0, sd1, sa0, sa1):
    _pass_a_body(h_hbm, sd_hbm, al_hbm, ar_hbm, alpha_hbm, m_hbm,
                 al_v, ar_v, m_v, idx_v, srows_v, drows_v, alpha_v,
                 red_v, mslice_v, m_sh, ss0, ss1, sd0, sd1, sa0, sa1)


def _pass_b_body(h_hbm, sd_hbm, alpha_hbm, m_hbm,
                 den_hbm, num_hbm,
                 m_v, idx_v, deff_v, srows_v,
                 alpha_v, exc_v, zs_v, den_sh, num_sh, ss0, ss1, sc0, sc1):
    cid = lax.axis_index("c")
    sid = lax.axis_index("s")
    wid = sid * 2 + cid
    iota = _iota16()
    sems = (ss0, ss1)
    sc_sems = (sc0, sc1)

    pltpu.sync_copy(m_hbm, m_v)

    def zrow_body(r, _):
        for j in range(D // 16):
            srows_v[0, r, pl.ds(j * 16, 16)] = jnp.zeros((16,), jnp.float32)
        return 0
    lax.fori_loop(0, CH_G, zrow_body, 0)
    for bb in range(SLICE // CH_G):
        pltpu.sync_copy(srows_v.at[0],
                        num_sh.at[pl.ds(sid * SLICE + bb * CH_G, CH_G)])

    def zden_body(w, _):
        zs_v[pl.ds(w * 16, 16)] = jnp.zeros((16,), jnp.float32)
        return 0
    lax.fori_loop(0, SLICE // 16, zden_body, 0)
    pltpu.sync_copy(zs_v, den_sh.at[pl.ds(sid * SLICE, SLICE)])
    plsc.subcore_barrier()

    def issue(c, b):
        base = wid * PER_TILE + c * CH_G
        pltpu.sync_copy(sd_hbm.at[wid * NCHUNK + c], idx_v.at[b])
        pltpu.sync_copy(alpha_hbm.at[pl.ds(base, CH_G)], alpha_v.at[b])
        pltpu.make_async_copy(h_hbm.at[idx_v.at[b, 0]], srows_v.at[b],
                              sems[b]).start()

    def wait(b):
        pltpu.make_async_copy(h_hbm.at[idx_v.at[b, 0]], srows_v.at[b],
                              sems[b]).wait()

    def compute(c, b):
        base = wid * PER_TILE + c * CH_G
        for g in range(CH_G // 16):
            src16 = idx_v[b, 0, pl.ds(g * 16, 16)]
            dst16 = idx_v[b, 1, pl.ds(g * 16, 16)]
            e16 = iota + (base + g * 16)
            deff = _dst_eff(src16, dst16, e16)
            deff_v[b, pl.ds(g * 16, 16)] = deff
            a16 = alpha_v[b, pl.ds(g * 16, 16)]
            ex = jnp.exp(a16 - plsc.load_gather(m_v, [deff]))
            exc_v[pl.ds(g * 16, 16)] = ex
            for l in range(16):
                exb = _perm(ex, _splat(l))
                r = g * 16 + l
                for j in range(D // 16):
                    sl = pl.ds(j * 16, 16)
                    srows_v[b, r, sl] = srows_v[b, r, sl] * exb

        pltpu.make_async_copy(srows_v.at[b], num_sh.at[deff_v.at[b]],
                              sc_sems[b]).start(add=True)
        pltpu.sync_copy(exc_v, den_sh.at[deff_v.at[b]], add=True)

    def scat_wait(b):
        pltpu.make_async_copy(srows_v.at[b], num_sh.at[deff_v.at[b]],
                              sc_sems[b]).wait()

    issue(0, 0)

    def piped(i, _):
        c0 = i * 2
        issue(c0 + 1, 1)
        wait(0)
        lax.cond(i > 0, lambda: scat_wait(0), lambda: None)
        compute(c0, 0)
        issue(c0 + 2, 0)
        wait(1)
        lax.cond(i > 0, lambda: scat_wait(1), lambda: None)
        compute(c0 + 1, 1)
        return 0
    lax.fori_loop(0, (NCHUNK - 1) // 2, piped, 0)
    wait(0)
    scat_wait(0)
    compute(NCHUNK - 1, 0)
    scat_wait(1)
    scat_wait(0)
    plsc.subcore_barrier()

    pltpu.sync_copy(den_sh.at[pl.ds(sid * SLICE, SLICE)],
                    den_hbm.at[cid, pl.ds(sid * SLICE, SLICE)])
    pltpu.sync_copy(num_sh.at[pl.ds(sid * SLICE, SLICE)],
                    num_hbm.at[cid, pl.ds(sid * SLICE, SLICE)])


@functools.partial(
    pl.kernel,
    out_type=(jax.ShapeDtypeStruct((2, NP), jnp.float32),
              jax.ShapeDtypeStruct((2, NP, D), jnp.float32)),
    mesh=_mesh,
    compiler_params=_sc_params,
    scratch_types=[
        pltpu.VMEM((NP,), jnp.float32),
        pltpu.VMEM((2, 2, CH_G), jnp.int32),
        pltpu.VMEM((2, CH_G), jnp.int32),
        pltpu.VMEM((2, CH_G, D), jnp.float32),
        pltpu.VMEM((2, CH_G), jnp.float32),
        pltpu.VMEM((CH_G,), jnp.float32),
        pltpu.VMEM((SLICE,), jnp.float32),
        pltpu.VMEM_SHARED((NP,), jnp.float32),
        pltpu.VMEM_SHARED((NP, D), jnp.float32),
        pltpu.SemaphoreType.DMA,
        pltpu.SemaphoreType.DMA,
        pltpu.SemaphoreType.DMA,
        pltpu.SemaphoreType.DMA,
    ],
)
def _pass_b(h_hbm, sd_hbm, alpha_hbm, m_hbm, den_hbm, num_hbm,
            m_v, idx_v, deff_v, srows_v, alpha_v,
            exc_v, zs_v, den_sh, num_sh, ss0, ss1, sc0, sc1):
    _pass_b_body(h_hbm, sd_hbm, alpha_hbm, m_hbm, den_hbm, num_hbm,
                 m_v, idx_v, deff_v, srows_v, alpha_v,
                 exc_v, zs_v, den_sh, num_sh, ss0, ss1, sc0, sc1)


def _tcmax_body(m_ref, o_ref):
    o_ref[...] = jnp.maximum(m_ref[0], m_ref[1])


def _tc_max(m):
    return pl.pallas_call(
        _tcmax_body,
        out_shape=jax.ShapeDtypeStruct((NP,), jnp.float32),
    )(m)


_RB = 1000


def _mmT(a, w):
    return lax.dot_general(a, w, (((1,), (1,)), ((), ())),
                           preferred_element_type=jnp.float32)


def _tc1_body(x_ref, w_ref, attl_ref, attr_ref, h_ref, al_ref, ar_ref):
    h = _mmT(x_ref[...], w_ref[...])
    h_ref[...] = h
    al_ref[...] = _mmT(h, attl_ref[...])
    ar_ref[...] = _mmT(h, attr_ref[...])


def _tc_dense1(x, W, att_l, att_r):
    return pl.pallas_call(
        _tc1_body,
        grid=(N // _RB,),
        in_specs=[
            pl.BlockSpec((_RB, D), lambda i: (i, 0)),
            pl.BlockSpec((D, D), lambda i: (0, 0)),
            pl.BlockSpec((1, D), lambda i: (0, 0)),
            pl.BlockSpec((1, D), lambda i: (0, 0)),
        ],
        out_specs=[
            pl.BlockSpec((_RB, D), lambda i: (i, 0)),
            pl.BlockSpec((_RB, 1), lambda i: (i, 0)),
            pl.BlockSpec((_RB, 1), lambda i: (i, 0)),
        ],
        out_shape=[
            jax.ShapeDtypeStruct((N, D), jnp.float32),
            jax.ShapeDtypeStruct((N, 1), jnp.float32),
            jax.ShapeDtypeStruct((N, 1), jnp.float32),
        ],
    )(x, W, att_l.reshape(1, D), att_r.reshape(1, D))


def _bn(x, g, b, m, v, eps=1e-5):
    return (x - m) / jnp.sqrt(v + eps) * g + b


def _tc2_body(num_ref, den_ref, b_ref, g_ref, bb_ref, m_ref, v_ref,
              w_ref, attl_ref, attr_ref, h_ref, al_ref, ar_ref):
    num = num_ref[0] + num_ref[1]
    den = den_ref[0, :, 0] + den_ref[1, :, 0]
    out = num / (den[:, None] + 1e-16) + b_ref[...]
    out = _bn(out, g_ref[...], bb_ref[...], m_ref[...], v_ref[...])
    hin = jnp.maximum(out, 0.0)
    h = _mmT(hin, w_ref[...])
    h_ref[...] = h
    al_ref[...] = _mmT(h, attl_ref[...])
    ar_ref[...] = _mmT(h, attr_ref[...])


def _tc_dense2(num, den, bias, bn_g, bn_b, bn_m, bn_v, W, att_l, att_r):
    vec = lambda a: a.reshape(1, D)
    return pl.pallas_call(
        _tc2_body,
        grid=(N // _RB,),
        in_specs=[
            pl.BlockSpec((2, _RB, D), lambda i: (0, i, 0)),
            pl.BlockSpec((2, _RB, 1), lambda i: (0, i, 0)),
        ] + [pl.BlockSpec((1, D), lambda i: (0, 0))] * 5 + [
            pl.BlockSpec((D, D), lambda i: (0, 0)),
            pl.BlockSpec((1, D), lambda i: (0, 0)),
            pl.BlockSpec((1, D), lambda i: (0, 0)),
        ],
        out_specs=[
            pl.BlockSpec((_RB, D), lambda i: (i, 0)),
            pl.BlockSpec((_RB, 1), lambda i: (i, 0)),
            pl.BlockSpec((_RB, 1), lambda i: (i, 0)),
        ],
        out_shape=[
            jax.ShapeDtypeStruct((N, D), jnp.float32),
            jax.ShapeDtypeStruct((N, 1), jnp.float32),
            jax.ShapeDtypeStruct((N, 1), jnp.float32),
        ],
    )(num, den.reshape(2, NP, 1), vec(bias), vec(bn_g), vec(bn_b),
      vec(bn_m), vec(bn_v), W, att_l.reshape(1, D), att_r.reshape(1, D))


def _tc3_body(num_ref, den_ref, b_ref, g2_ref, b2_ref, m2_ref, v2_ref,
              f1w_ref, f1b_ref, g3_ref, b3_ref, m3_ref, v3_ref,
              f2w_ref, f2b_ref, g4_ref, b4_ref, m4_ref, v4_ref,
              f3w_ref, f3b_ref, o_ref):
    num = num_ref[0] + num_ref[1]
    den = den_ref[0, :, 0] + den_ref[1, :, 0]
    out = num / (den[:, None] + 1e-16) + b_ref[...]
    h = jnp.maximum(_bn(out, g2_ref[...], b2_ref[...], m2_ref[...],
                        v2_ref[...]), 0.0)
    h = jnp.maximum(_bn(_mmT(h, f1w_ref[...]) + f1b_ref[...],
                        g3_ref[...], b3_ref[...], m3_ref[...], v3_ref[...]),
                    0.0)
    h = jnp.maximum(_bn(_mmT(h, f2w_ref[...]) + f2b_ref[...],
                        g4_ref[...], b4_ref[...], m4_ref[...], v4_ref[...]),
                    0.0)
    o_ref[...] = _mmT(h, f3w_ref[...]) + f3b_ref[...]


def _tc_dense3(num, den, b2, bn2_g, bn2_b, bn2_m, bn2_v,
               fc1_w, fc1_b, bn3_g, bn3_b, bn3_m, bn3_v,
               fc2_w, fc2_b, bn4_g, bn4_b, bn4_m, bn4_v, fc3_w, fc3_b):
    vec = lambda a: a.reshape(1, -1)
    return pl.pallas_call(
        _tc3_body,
        grid=(N // _RB,),
        in_specs=[
            pl.BlockSpec((2, _RB, D), lambda i: (0, i, 0)),
            pl.BlockSpec((2, _RB, 1), lambda i: (0, i, 0)),
        ] + [pl.BlockSpec((1, D), lambda i: (0, 0))] * 5 + [
            pl.BlockSpec((D, D), lambda i: (0, 0)),
        ] + [pl.BlockSpec((1, D), lambda i: (0, 0))] * 5 + [
            pl.BlockSpec((D, D), lambda i: (0, 0)),
        ] + [pl.BlockSpec((1, D), lambda i: (0, 0))] * 5 + [
            pl.BlockSpec((2, D), lambda i: (0, 0)),
            pl.BlockSpec((1, 2), lambda i: (0, 0)),
        ],
        out_specs=[pl.BlockSpec((_RB, 2), lambda i: (i, 0))],
        out_shape=[jax.ShapeDtypeStruct((N, 2), jnp.float32)],
    )(num, den.reshape(2, NP, 1), vec(b2), vec(bn2_g), vec(bn2_b),
      vec(bn2_m), vec(bn2_v), fc1_w, vec(fc1_b), vec(bn3_g), vec(bn3_b),
      vec(bn3_m), vec(bn3_v), fc2_w, vec(fc2_b), vec(bn4_g), vec(bn4_b),
      vec(bn4_m), vec(bn4_v), fc3_w, vec(fc3_b))[0]


def kernel(x, edge_index, W1, att_l1, att_r1, b1, bn1_g, bn1_b, bn1_m, bn1_v,
           W2, att_l2, att_r2, b2, bn2_g, bn2_b, bn2_m, bn2_v,
           fc1_w, fc1_b, bn3_g, bn3_b, bn3_m, bn3_v,
           fc2_w, fc2_b, bn4_g, bn4_b, bn4_m, bn4_v, fc3_w, fc3_b):
    loops = jnp.arange(N, dtype=jnp.int32)
    pad = jnp.zeros((EP - EF,), jnp.int32)
    src = jnp.concatenate([edge_index[0], loops, pad])
    dst = jnp.concatenate([edge_index[1], loops, pad])
    sd = jnp.stack([src.reshape(-1, CH_G), dst.reshape(-1, CH_G)], axis=1)

    h1, al1, ar1 = _tc_dense1(x, W1, att_l1, att_r1)
    alpha1, m1 = _pass_a(h1, sd, al1.reshape(N), ar1.reshape(N))
    den1, num1 = _pass_b(h1, sd, alpha1, _tc_max(m1))
    h2, al2, ar2 = _tc_dense2(num1, den1, b1, bn1_g, bn1_b, bn1_m, bn1_v,
                              W2, att_l2, att_r2)
    alpha2, m2 = _pass_a(h2, sd, al2.reshape(N), ar2.reshape(N))
    den2, num2 = _pass_b(h2, sd, alpha2, _tc_max(m2))
    return _tc_dense3(num2, den2, b2, bn2_g, bn2_b, bn2_m, bn2_v,
                      fc1_w, fc1_b, bn3_g, bn3_b, bn3_m, bn3_v,
                      fc2_w, fc2_b, bn4_g, bn4_b, bn4_m, bn4_v,
                      fc3_w, fc3_b)
